# baseline (speedup 1.0000x reference)
import functools

import jax
import jax.numpy as jnp
from jax import lax
from jax.experimental import pallas as pl
from jax.experimental.pallas import tpu as pltpu
from jax.experimental.pallas import tpu_sc as plsc

DIM = 32


@functools.partial(jax.jit, static_argnums=(2,))
def _sc_gather(idx_t, table, n_workers):
    seq, b_total = idx_t.shape
    b_pw = b_total // n_workers

    mesh = plsc.VectorSubcoreMesh(core_axis_name="c", subcore_axis_name="s")

    @functools.partial(
        pl.kernel,
        mesh=mesh,
        out_type=jax.ShapeDtypeStruct((seq, b_total, DIM), jnp.float32),
        scratch_types=[
            pltpu.VMEM((seq, b_pw), jnp.int32),
            pltpu.VMEM((2, b_pw, DIM), jnp.float32),
            pltpu.SemaphoreType.DMA((2,)),
        ],
        compiler_params=pltpu.CompilerParams(use_tc_tiling_on_sc=False),
    )
    def k(idx_hbm, table_hbm, out_hbm, idx_v, rows_v, sems):
        n_cores = 2
        wid = lax.axis_index("s") * n_cores + lax.axis_index("c")
        base = wid * b_pw
        pltpu.sync_copy(idx_hbm.at[:, pl.ds(base, b_pw)], idx_v)

        def start(s, buf):
            pltpu.make_async_copy(
                table_hbm.at[idx_v.at[s]], rows_v.at[buf], sems.at[buf]
            ).start()

        def wait(buf):
            pltpu.make_async_copy(
                table_hbm.at[idx_v.at[0]], rows_v.at[buf], sems.at[buf]
            ).wait()

        start(0, 0)

        def body(sp, _):
            for buf in range(2):
                s = sp * 2 + buf
                nxt = s + 1

                @pl.when(nxt < seq)
                def _():
                    start(nxt, 1 - buf)

                wait(buf)
                pltpu.sync_copy(
                    rows_v.at[buf], out_hbm.at[s, pl.ds(base, b_pw)]
                )
            return 0

        lax.fori_loop(0, seq // 2, body, 0, unroll=False)

    return k(idx_t, table)


def _tc_transpose_block(x_ref, o_ref):
    x = x_ref[...]
    o = jnp.transpose(x.reshape(32, 4, 32), (2, 0, 1)).reshape(4, 8, 128)
    o_ref[0, :, 0] = o


@jax.jit
def _tc_transpose(xp):
    seq, b_total, _ = xp.shape
    n_bt = b_total // 128
    x2 = xp.reshape(seq * b_total * DIM // 128, 128)
    return pl.pallas_call(
        _tc_transpose_block,
        grid=(seq, n_bt),
        in_specs=[pl.BlockSpec((32, 128), lambda s, t: (s * n_bt + t, 0))],
        out_specs=pl.BlockSpec(
            (1, DIM // 8, 1, 8, 128), lambda s, t: (s, 0, t, 0, 0)
        ),
        out_shape=jax.ShapeDtypeStruct(
            (seq, DIM // 8, n_bt, 8, 128), jnp.float32
        ),
    )(x2)


def kernel(token_ids, embedding_matrix):
    b, s = token_ids.shape
    idx_t = token_ids.T.astype(jnp.int32)
    xp = _sc_gather(idx_t, embedding_matrix, 32)
    o5 = _tc_transpose(xp)
    return o5.transpose(2, 4, 0, 1, 3).reshape(b, s, DIM)


# speedup vs baseline: 3.0536x; 3.0536x over previous
import functools

import jax
import jax.numpy as jnp
from jax import lax
from jax.experimental import pallas as pl
from jax.experimental.pallas import tpu as pltpu
from jax.experimental.pallas import tpu_sc as plsc

DIM = 32
LANES = 128


@functools.partial(jax.jit, static_argnums=(2,))
def _sc_gather(idx_t, table, n_workers):
    seq, b_total = idx_t.shape
    b_pw = b_total // n_workers

    mesh = plsc.VectorSubcoreMesh(core_axis_name="c", subcore_axis_name="s")

    @functools.partial(
        pl.kernel,
        mesh=mesh,
        out_type=jax.ShapeDtypeStruct((seq, b_total, LANES), jnp.float32),
        scratch_types=[
            pltpu.VMEM((seq, b_pw), jnp.int32),
            pltpu.VMEM((2, b_pw, DIM), jnp.float32),
            pltpu.SemaphoreType.DMA((2,)),
        ],
        compiler_params=pltpu.CompilerParams(use_tc_tiling_on_sc=False),
    )
    def k(idx_hbm, table_hbm, out_hbm, idx_v, rows_v, sems):
        n_cores = 2
        wid = lax.axis_index("s") * n_cores + lax.axis_index("c")
        base = wid * b_pw
        pltpu.sync_copy(idx_hbm.at[:, pl.ds(base, b_pw)], idx_v)

        def start(s, buf):
            pltpu.make_async_copy(
                table_hbm.at[idx_v.at[s]], rows_v.at[buf], sems.at[buf]
            ).start()

        def wait(buf):
            pltpu.make_async_copy(
                table_hbm.at[idx_v.at[0]], rows_v.at[buf], sems.at[buf]
            ).wait()

        start(0, 0)

        def body(sp, _):
            for buf in range(2):
                s = sp * 2 + buf
                nxt = s + 1

                @pl.when(nxt < seq)
                def _():
                    start(nxt, 1 - buf)

                wait(buf)
                pltpu.sync_copy(
                    rows_v.at[buf],
                    out_hbm.at[s, pl.ds(base, b_pw), pl.ds(0, DIM)],
                )
            return 0

        lax.fori_loop(0, seq // 2, body, 0, unroll=False)

    return k(idx_t, table)


def _tc_transpose_block(x_ref, o_ref):
    t = jnp.transpose(x_ref[0], (1, 0))
    o_ref[0] = t[:DIM]


@jax.jit
def _tc_transpose(xp):
    seq, b_total, _ = xp.shape
    n_bt = b_total // 512
    return pl.pallas_call(
        _tc_transpose_block,
        grid=(seq, n_bt),
        in_specs=[pl.BlockSpec((1, 512, LANES), lambda s, t: (s, t, 0))],
        out_specs=pl.BlockSpec((1, DIM, 512), lambda s, t: (s, 0, t)),
        out_shape=jax.ShapeDtypeStruct((seq, DIM, b_total), jnp.float32),
    )(xp)


def kernel(token_ids, embedding_matrix):
    b, s = token_ids.shape
    idx_t = token_ids.T.astype(jnp.int32)
    xp = _sc_gather(idx_t, embedding_matrix, 32)
    o3 = _tc_transpose(xp)
    return o3.transpose(2, 0, 1).reshape(b, s, DIM)


# speedup vs baseline: 4.5212x; 1.4806x over previous
import functools

import jax
import jax.numpy as jnp
from jax import lax
from jax.experimental import pallas as pl
from jax.experimental.pallas import tpu as pltpu
from jax.experimental.pallas import tpu_sc as plsc

DIM = 32
GROUP_B = 16


@functools.partial(jax.jit, static_argnums=(2,))
def _sc_gather(idx3, table, n_workers):
    _, b_pw, seq = idx3.shape
    n_groups = b_pw // GROUP_B

    mesh = plsc.VectorSubcoreMesh(core_axis_name="c", subcore_axis_name="s")

    @functools.partial(
        pl.kernel,
        mesh=mesh,
        out_type=jax.ShapeDtypeStruct((n_workers * b_pw, seq, DIM), jnp.float32),
        scratch_types=[
            pltpu.VMEM((b_pw, seq), jnp.int32),
            pltpu.VMEM((2, GROUP_B, seq, DIM), jnp.float32),
            pltpu.SemaphoreType.DMA((2,)),
        ],
        compiler_params=pltpu.CompilerParams(use_tc_tiling_on_sc=False),
    )
    def k(idx_hbm, table_hbm, out_hbm, idx_v, rows_v, sems):
        n_cores = 2
        wid = lax.axis_index("s") * n_cores + lax.axis_index("c")
        base = wid * b_pw
        pltpu.sync_copy(idx_hbm.at[wid], idx_v)

        def start_group(g, buf):
            for i in range(GROUP_B):
                pltpu.make_async_copy(
                    table_hbm.at[idx_v.at[g * GROUP_B + i]],
                    rows_v.at[buf, i],
                    sems.at[buf],
                ).start()

        def wait_group(buf):
            for i in range(GROUP_B):
                pltpu.make_async_copy(
                    table_hbm.at[idx_v.at[0]],
                    rows_v.at[buf, i],
                    sems.at[buf],
                ).wait()

        start_group(0, 0)

        def body(gp, _):
            for buf in range(2):
                g = gp * 2 + buf
                nxt = g + 1

                @pl.when(nxt < n_groups)
                def _():
                    start_group(nxt, 1 - buf)

                wait_group(buf)
                pltpu.sync_copy(
                    rows_v.at[buf],
                    out_hbm.at[pl.ds(base + g * GROUP_B, GROUP_B)],
                )
            return 0

        lax.fori_loop(0, n_groups // 2, body, 0, unroll=False)

    return k(idx3, table)


def kernel(token_ids, embedding_matrix):
    b, s = token_ids.shape
    n_workers = 32
    b_pw = b // n_workers
    idx3 = token_ids.reshape(n_workers, b_pw, s).astype(jnp.int32)
    return _sc_gather(idx3, embedding_matrix, n_workers)
